# baseline (speedup 1.0000x reference)
import jax
import jax.numpy as jnp
from jax import lax
from jax.experimental import pallas as pl
from jax.experimental.pallas import tpu as pltpu

_NC = 8192
_G = 8
_THIRD = 1.0 / 3.0


def _one_batch(pc, feat, tm, out_ref, g):
    n = pc.shape[1]

    tmb = tm.astype(jnp.bfloat16).astype(jnp.float32)
    pcb = pc.astype(jnp.bfloat16).astype(jnp.float32)
    e = (tmb[:, 0:1] * pcb[0:1, :]
         + tmb[:, 1:2] * pcb[1:2, :]
         + tmb[:, 2:3] * pcb[2:3, :])

    g_f = jnp.round(e * _THIRD) * 3.0
    d = e - g_f
    d0, d1, d2 = d[0:1, :], d[1:2, :], d[2:3, :]
    r0 = (d1 > d0).astype(jnp.int32) + (d2 > d0).astype(jnp.int32)
    r1 = (d0 >= d1).astype(jnp.int32) + (d2 > d1).astype(jnp.int32)

    g0, g1, g2 = g_f[0:1, :], g_f[1:2, :], g_f[2:3, :]
    rs = (((g0 + g1) + g2) * _THIRD).astype(jnp.int32)
    pos = rs > 0
    neg = rs < 0

    def adjust(gf, ri):
        cond = (pos & (ri >= 3 - rs)) | (neg & (ri < -rs))
        shift = jnp.where(cond, jnp.where(pos, -3, 3), 0)
        gint = gf.astype(jnp.int32) + shift
        rnew = ri + shift + rs
        return gint, jnp.clip(rnew, 0, 2)

    gi0, rc0 = adjust(g0, r0)
    gi1, rc1 = adjust(g1, r1)

    off0 = jnp.min(gi0 - rc0, keepdims=True)
    off1 = jnp.min(gi1 - rc1, keepdims=True)
    v0 = gi0 - (off0 + (-off0) % 3)
    v1 = gi1 - (off1 + (-off1) % 3)

    iota3 = lax.broadcasted_iota(jnp.int32, (128, _NC), 0) * 3

    accs = [jnp.zeros((128, 128), jnp.float32) for _ in range(3)]
    for k in range(n // _NC):
        sl = slice(k * _NC, (k + 1) * _NC)
        mr = iota3 == v0[:, sl]
        mc = iota3 == v1[:, sl]
        cmat = jnp.where(mc, 1.0, 0.0)
        for f in range(3):
            amat = jnp.where(mr, feat[f:f + 1, sl], 0.0)
            accs[f] = accs[f] + lax.dot_general(
                amat, cmat, (((1,), (1,)), ((), ())),
                preferred_element_type=jnp.float32)

    for f in range(3):
        out_ref[g, f] = accs[f]


def _splat_body(pc_ref, feat_ref, tm_ref, out_ref):
    for g in range(_G):
        _one_batch(pc_ref[g], feat_ref[g], tm_ref[g], out_ref, g)


def _run(pc1, features, trans_mat):
    B, _, N = pc1.shape
    return pl.pallas_call(
        _splat_body,
        grid=(B // _G,),
        in_specs=[
            pl.BlockSpec((_G, 3, N), lambda b: (b, 0, 0)),
            pl.BlockSpec((_G, 3, N), lambda b: (b, 0, 0)),
            pl.BlockSpec((_G, 3, 3), lambda b: (b, 0, 0)),
        ],
        out_specs=pl.BlockSpec((_G, 3, 128, 128), lambda b: (b, 0, 0, 0)),
        out_shape=jax.ShapeDtypeStruct((B, 3, 128, 128), jnp.float32),
        compiler_params=pltpu.CompilerParams(
            dimension_semantics=("parallel",),
        ),
        name="lattice_splat",
    )(pc1, features, trans_mat)


def kernel(pc1, features, trans_mat):
    out = _run(pc1, features, trans_mat)
    return jnp.transpose(out, (0, 2, 3, 1))


# speedup vs baseline: 1.0799x; 1.0799x over previous
import jax
import jax.numpy as jnp
from jax import lax
from jax.experimental import pallas as pl
from jax.experimental.pallas import tpu as pltpu

_NC = 8192
_G = 8
_THIRD = 1.0 / 3.0


def _one_batch(pc, feat, tm, out_ref, g):
    n = pc.shape[1]

    tmb = tm.astype(jnp.bfloat16).astype(jnp.float32)
    pcb = pc.astype(jnp.bfloat16).astype(jnp.float32)
    e = (tmb[:, 0:1] * pcb[0:1, :]
         + tmb[:, 1:2] * pcb[1:2, :]
         + tmb[:, 2:3] * pcb[2:3, :])

    g_f = jnp.round(e * _THIRD) * 3.0
    d = e - g_f
    d0, d1, d2 = d[0:1, :], d[1:2, :], d[2:3, :]
    r0 = (d1 > d0).astype(jnp.int32) + (d2 > d0).astype(jnp.int32)
    r1 = (d0 >= d1).astype(jnp.int32) + (d2 > d1).astype(jnp.int32)

    g0, g1, g2 = g_f[0:1, :], g_f[1:2, :], g_f[2:3, :]
    rs = (((g0 + g1) + g2) * _THIRD).astype(jnp.int32)
    pos = rs > 0
    neg = rs < 0

    r01 = jnp.concatenate([r0, r1], axis=0)
    g01 = g_f[0:2, :]
    cond = (pos & (r01 >= 3 - rs)) | (neg & (r01 < -rs))
    shift = jnp.where(cond, jnp.where(pos, -3, 3), 0)
    gi = g01.astype(jnp.int32) + shift
    rc = jnp.clip(r01 + shift + rs, 0, 2)

    off = jnp.min(gi - rc, axis=1, keepdims=True)
    v = gi - (off + (-off) % 3)
    v0 = v[0:1, :]
    v1 = v[1:2, :]

    iota3 = lax.broadcasted_iota(jnp.int32, (128, _NC), 0) * 3

    accs = [jnp.zeros((128, 128), jnp.float32) for _ in range(3)]
    for k in range(n // _NC):
        sl = slice(k * _NC, (k + 1) * _NC)
        mr = iota3 == v0[:, sl]
        mc = iota3 == v1[:, sl]
        cmat = jnp.where(mc, 1.0, 0.0)
        for f in range(3):
            amat = jnp.where(mr, feat[f:f + 1, sl], 0.0)
            accs[f] = accs[f] + lax.dot_general(
                amat, cmat, (((1,), (1,)), ((), ())),
                preferred_element_type=jnp.float32)

    for f in range(3):
        out_ref[g, f] = accs[f]


def _splat_body(pc_ref, feat_ref, tm_ref, out_ref):
    for g in range(_G):
        _one_batch(pc_ref[g], feat_ref[g], tm_ref[g], out_ref, g)


def _run(pc1, features, trans_mat):
    B, _, N = pc1.shape
    return pl.pallas_call(
        _splat_body,
        grid=(B // _G,),
        in_specs=[
            pl.BlockSpec((_G, 3, N), lambda b: (b, 0, 0)),
            pl.BlockSpec((_G, 3, N), lambda b: (b, 0, 0)),
            pl.BlockSpec((_G, 3, 3), lambda b: (b, 0, 0)),
        ],
        out_specs=pl.BlockSpec((_G, 3, 128, 128), lambda b: (b, 0, 0, 0)),
        out_shape=jax.ShapeDtypeStruct((B, 3, 128, 128), jnp.float32),
        compiler_params=pltpu.CompilerParams(
            dimension_semantics=("parallel",),
        ),
        name="lattice_splat",
    )(pc1, features, trans_mat)


def kernel(pc1, features, trans_mat):
    out = _run(pc1, features, trans_mat)
    return jnp.transpose(out, (0, 2, 3, 1))
